# baseline (device time: 58960 ns/iter reference)
import jax
import jax.numpy as jnp
from jax import lax
from jax.experimental import pallas as pl
from jax.experimental.pallas import tpu as pltpu

_CHUNK_ROWS = [32] * 4 + [112] * 16 + [32] * 4
N_CHUNKS = len(_CHUNK_ROWS)
_OFFS = [sum(_CHUNK_ROWS[:c]) for c in range(N_CHUNKS)]


def kernel(x):
    M, N = x.shape
    H = M // 2
    assert sum(_CHUNK_ROWS) == H

    def body(x_ref, out_ref, stage, xsend, xrecv, ssum,
             l_sems, o_sems, xs_sems, xr_sems, ys_sems, yr_sems):
        mx = lax.axis_index("x")
        my = lax.axis_index("y")
        x_peer = (1 - mx, my)
        y_peer = (mx, 1 - my)

        base = my * H

        loads = []
        for c in range(N_CHUNKS):
            r = pl.ds(_OFFS[c], _CHUNK_ROWS[c])
            cp = pltpu.make_async_copy(
                x_ref.at[pl.ds(base + _OFFS[c], _CHUNK_ROWS[c]), :],
                stage.at[r, :],
                l_sems.at[c],
            )
            cp.start()
            loads.append(cp)
        r0 = pl.ds(_OFFS[0], _CHUNK_ROWS[0])
        loads[0].wait()
        xsend[r0, :] = stage[r0, :].astype(jnp.bfloat16)

        barrier = pltpu.get_barrier_semaphore()
        for nbr in (x_peer, y_peer):
            pl.semaphore_signal(barrier, inc=1, device_id=nbr,
                                device_id_type=pl.DeviceIdType.MESH)
        pl.semaphore_wait(barrier, 2)

        p1 = []
        for c in range(N_CHUNKS):
            r = pl.ds(_OFFS[c], _CHUNK_ROWS[c])
            if c > 0:
                loads[c].wait()
                xsend[r, :] = stage[r, :].astype(jnp.bfloat16)
            rdma = pltpu.make_async_remote_copy(
                src_ref=xsend.at[r, :],
                dst_ref=xrecv.at[r, :],
                send_sem=xs_sems.at[c],
                recv_sem=xr_sems.at[c],
                device_id=x_peer,
                device_id_type=pl.DeviceIdType.MESH,
            )
            rdma.start()
            p1.append(rdma)

        p2 = []
        ostores = []
        for c in range(N_CHUNKS):
            r = pl.ds(_OFFS[c], _CHUNK_ROWS[c])
            p1[c].wait_recv()
            rows = pl.ds(base + _OFFS[c], _CHUNK_ROWS[c])
            ssum[r, :] = xsend[r, :] + xrecv[r, :]
            rdma = pltpu.make_async_remote_copy(
                src_ref=ssum.at[r, :],
                dst_ref=out_ref.at[rows, :],
                send_sem=ys_sems.at[c],
                recv_sem=yr_sems.at[c],
                device_id=y_peer,
                device_id_type=pl.DeviceIdType.MESH,
            )
            rdma.start()
            p2.append(rdma)
            ocp = pltpu.make_async_copy(
                ssum.at[r, :], out_ref.at[rows, :], o_sems.at[c],
            )
            ocp.start()
            ostores.append(ocp)

        for c in range(N_CHUNKS):
            p1[c].wait_send()
            p2[c].wait_send()
            p2[c].wait_recv()
            ostores[c].wait()

    out_shape = jax.ShapeDtypeStruct((M, N), jnp.bfloat16)
    return pl.pallas_call(
        body,
        out_shape=out_shape,
        in_specs=[pl.BlockSpec(memory_space=pl.ANY)],
        out_specs=pl.BlockSpec(memory_space=pl.ANY),
        scratch_shapes=[
            pltpu.VMEM((H, N), jnp.float32),
            pltpu.VMEM((H, N), jnp.bfloat16),
            pltpu.VMEM((H, N), jnp.bfloat16),
            pltpu.VMEM((H, N), jnp.bfloat16),
            pltpu.SemaphoreType.DMA((N_CHUNKS,)),
            pltpu.SemaphoreType.DMA((N_CHUNKS,)),
            pltpu.SemaphoreType.DMA((N_CHUNKS,)),
            pltpu.SemaphoreType.DMA((N_CHUNKS,)),
            pltpu.SemaphoreType.DMA((N_CHUNKS,)),
            pltpu.SemaphoreType.DMA((N_CHUNKS,)),
        ],
        compiler_params=pltpu.CompilerParams(collective_id=0),
    )(x)


# device time: 58404 ns/iter; 1.0095x vs baseline; 1.0095x over previous
import jax
import jax.numpy as jnp
from jax import lax
from jax.experimental import pallas as pl
from jax.experimental.pallas import tpu as pltpu

_CHUNK_ROWS = [64] * 32
N_CHUNKS = len(_CHUNK_ROWS)
_OFFS = [sum(_CHUNK_ROWS[:c]) for c in range(N_CHUNKS)]


def kernel(x):
    M, N = x.shape
    H = M // 2
    assert sum(_CHUNK_ROWS) == H

    def body(x_ref, out_ref, stage, xsend, xrecv, ssum,
             l_sems, o_sems, xs_sems, xr_sems, ys_sems, yr_sems):
        mx = lax.axis_index("x")
        my = lax.axis_index("y")
        x_peer = (1 - mx, my)
        y_peer = (mx, 1 - my)

        base = my * H

        loads = []
        for c in range(N_CHUNKS):
            r = pl.ds(_OFFS[c], _CHUNK_ROWS[c])
            cp = pltpu.make_async_copy(
                x_ref.at[pl.ds(base + _OFFS[c], _CHUNK_ROWS[c]), :],
                stage.at[r, :],
                l_sems.at[c],
            )
            cp.start()
            loads.append(cp)
        r0 = pl.ds(_OFFS[0], _CHUNK_ROWS[0])
        loads[0].wait()
        xsend[r0, :] = stage[r0, :].astype(jnp.bfloat16)

        barrier = pltpu.get_barrier_semaphore()
        for nbr in (x_peer, y_peer):
            pl.semaphore_signal(barrier, inc=1, device_id=nbr,
                                device_id_type=pl.DeviceIdType.MESH)
        pl.semaphore_wait(barrier, 2)

        p1 = []
        for c in range(N_CHUNKS):
            r = pl.ds(_OFFS[c], _CHUNK_ROWS[c])
            if c > 0:
                loads[c].wait()
                xsend[r, :] = stage[r, :].astype(jnp.bfloat16)
            rdma = pltpu.make_async_remote_copy(
                src_ref=xsend.at[r, :],
                dst_ref=xrecv.at[r, :],
                send_sem=xs_sems.at[c],
                recv_sem=xr_sems.at[c],
                device_id=x_peer,
                device_id_type=pl.DeviceIdType.MESH,
            )
            rdma.start()
            p1.append(rdma)

        p2 = []
        ostores = []
        for c in range(N_CHUNKS):
            r = pl.ds(_OFFS[c], _CHUNK_ROWS[c])
            p1[c].wait_recv()
            rows = pl.ds(base + _OFFS[c], _CHUNK_ROWS[c])
            ssum[r, :] = xsend[r, :] + xrecv[r, :]
            rdma = pltpu.make_async_remote_copy(
                src_ref=ssum.at[r, :],
                dst_ref=out_ref.at[rows, :],
                send_sem=ys_sems.at[c],
                recv_sem=yr_sems.at[c],
                device_id=y_peer,
                device_id_type=pl.DeviceIdType.MESH,
            )
            rdma.start()
            p2.append(rdma)
            ocp = pltpu.make_async_copy(
                ssum.at[r, :], out_ref.at[rows, :], o_sems.at[c],
            )
            ocp.start()
            ostores.append(ocp)

        for c in range(N_CHUNKS):
            p1[c].wait_send()
            p2[c].wait_send()
            p2[c].wait_recv()
            ostores[c].wait()

    out_shape = jax.ShapeDtypeStruct((M, N), jnp.bfloat16)
    return pl.pallas_call(
        body,
        out_shape=out_shape,
        in_specs=[pl.BlockSpec(memory_space=pl.ANY)],
        out_specs=pl.BlockSpec(memory_space=pl.ANY),
        scratch_shapes=[
            pltpu.VMEM((H, N), jnp.float32),
            pltpu.VMEM((H, N), jnp.bfloat16),
            pltpu.VMEM((H, N), jnp.bfloat16),
            pltpu.VMEM((H, N), jnp.bfloat16),
            pltpu.SemaphoreType.DMA((N_CHUNKS,)),
            pltpu.SemaphoreType.DMA((N_CHUNKS,)),
            pltpu.SemaphoreType.DMA((N_CHUNKS,)),
            pltpu.SemaphoreType.DMA((N_CHUNKS,)),
            pltpu.SemaphoreType.DMA((N_CHUNKS,)),
            pltpu.SemaphoreType.DMA((N_CHUNKS,)),
        ],
        compiler_params=pltpu.CompilerParams(collective_id=0),
    )(x)
